# Initial kernel scaffold; baseline (speedup 1.0000x reference)
import jax
import jax.numpy as jnp
from jax.experimental import pallas as pl


def kernel(x, edge_index, W1_0, b1_0, W2_0, b2_0, W1_1, b1_1, W2_1, b2_1):
    raise NotImplementedError("write your pallas kernel here")


# speedup vs baseline: 1.9707x; 1.9707x over previous
import functools

import jax
import jax.numpy as jnp
from jax import lax
from jax.experimental import pallas as pl
from jax.experimental.pallas import tpu as pltpu
from jax.experimental.pallas import tpu_sc as plsc

N = 10000
E = 320000
D = 128

NC = 2
NS = 16
NW = NC * NS
LANES = 16
VPR = D // LANES

EPW = E // NW
G_BLK = 80
G_NBLK = EPW // G_BLK

_mesh = plsc.VectorSubcoreMesh(
    core_axis_name="c", subcore_axis_name="s", num_cores=NC, num_subcores=NS
)


def _worker_id():
    return lax.axis_index("s") * NC + lax.axis_index("c")


def _ab_body(h_ref, wa_ref, wb_ref, b1_ref, a_ref, b_ref):
    h = h_ref[...]
    a_ref[...] = (
        jnp.dot(h, wa_ref[...], preferred_element_type=jnp.float32) + b1_ref[...]
    )
    b_ref[...] = jnp.dot(h, wb_ref[...], preferred_element_type=jnp.float32)


def _tc_ab(h, wa_t, wb_t, b1r):
    blk = 1000
    return pl.pallas_call(
        _ab_body,
        grid=(N // blk,),
        in_specs=[
            pl.BlockSpec((blk, D), lambda i: (i, 0)),
            pl.BlockSpec((D, D), lambda i: (0, 0)),
            pl.BlockSpec((D, D), lambda i: (0, 0)),
            pl.BlockSpec((1, D), lambda i: (0, 0)),
        ],
        out_specs=[
            pl.BlockSpec((blk, D), lambda i: (i, 0)),
            pl.BlockSpec((blk, D), lambda i: (i, 0)),
        ],
        out_shape=[
            jax.ShapeDtypeStruct((N, D), jnp.float32),
            jax.ShapeDtypeStruct((N, D), jnp.float32),
        ],
    )(h, wa_t, wb_t, b1r)


def _mlp2_body(m_ref, w2_ref, b2_ref, o_ref):
    o_ref[...] = (
        jnp.dot(m_ref[...], w2_ref[...], preferred_element_type=jnp.float32)
        + b2_ref[...]
    )


def _tc_mlp2(m, w2_t, b2r):
    blk = 2000
    return pl.pallas_call(
        _mlp2_body,
        grid=(E // blk,),
        in_specs=[
            pl.BlockSpec((blk, D), lambda i: (i, 0)),
            pl.BlockSpec((D, D), lambda i: (0, 0)),
            pl.BlockSpec((1, D), lambda i: (0, 0)),
        ],
        out_specs=pl.BlockSpec((blk, D), lambda i: (i, 0)),
        out_shape=jax.ShapeDtypeStruct((E, D), jnp.float32),
    )(m, w2_t, b2r)


@functools.partial(
    pl.kernel,
    out_type=jax.ShapeDtypeStruct((E, D), jnp.float32),
    mesh=_mesh,
    scratch_types=[
        pltpu.VMEM((EPW,), jnp.int32),
        pltpu.VMEM((EPW,), jnp.int32),
        pltpu.VMEM((G_BLK, D), jnp.float32),
        pltpu.VMEM((G_BLK, D), jnp.float32),
        pltpu.SemaphoreType.DMA,
        pltpu.SemaphoreType.DMA,
    ],
)
def _sc_gather_relu(a_hbm, b_hbm, dst_hbm, src_hbm, m_hbm, dstv, srcv, arows, brows, sem_a, sem_b):
    wid = _worker_id()
    ebase = wid * EPW
    pltpu.sync_copy(dst_hbm.at[pl.ds(ebase, EPW)], dstv)
    pltpu.sync_copy(src_hbm.at[pl.ds(ebase, EPW)], srcv)

    def blk_body(i, _):
        off = i * G_BLK
        cp_a = pltpu.async_copy(a_hbm.at[dstv.at[pl.ds(off, G_BLK)]], arows, sem_a)
        cp_b = pltpu.async_copy(b_hbm.at[srcv.at[pl.ds(off, G_BLK)]], brows, sem_b)
        cp_a.wait()
        cp_b.wait()

        def row_body(r, _):
            for c in range(VPR):
                sl = pl.ds(c * LANES, LANES)
                arows[r, sl] = jnp.maximum(arows[r, sl] + brows[r, sl], 0.0)
            return ()

        lax.fori_loop(0, G_BLK, row_body, ())
        pltpu.sync_copy(arows, m_hbm.at[pl.ds(ebase + off, G_BLK)])
        return ()

    lax.fori_loop(0, G_NBLK, blk_body, ())


def kernel(x, edge_index, W1_0, b1_0, W2_0, b2_0, W1_1, b1_1, W2_1, b2_1):
    src = edge_index[0]
    dst = edge_index[1]
    h = x
    for W1, b1, W2, b2 in (
        (W1_0, b1_0, W2_0, b2_0),
        (W1_1, b1_1, W2_1, b2_1),
    ):
        wa_t = jnp.transpose(W1[:, :D] - W1[:, D:])
        wb_t = jnp.transpose(W1[:, D:])
        w2_t = jnp.transpose(W2)
        a, b = _tc_ab(h, wa_t, wb_t, b1.reshape(1, D))
        m = _sc_gather_relu(a, b, dst, src)
        h2 = _tc_mlp2(m, w2_t, b2.reshape(1, D))
        h = jnp.zeros((N, D), jnp.float32).at[dst].max(h2)
    return h
